# Initial kernel scaffold; baseline (speedup 1.0000x reference)
import jax
import jax.numpy as jnp
from jax.experimental import pallas as pl


def kernel(x, W_dense, b_dense, W_edge1, b_edge1, W_root1, bias1, bn1_g, bn1_b, W_edge2, b_edge2, W_root2, bias2, bn2_g, bn2_b):
    raise NotImplementedError("write your pallas kernel here")


# speedup vs baseline: 1.5548x; 1.5548x over previous
import functools
import numpy as np
import jax
import jax.numpy as jnp
from jax import lax
from jax.experimental import pallas as pl
from jax.experimental.pallas import tpu as pltpu

GB = 1024
GNH = 128
GLD = 128
GH0 = 16
GH1 = 16
GNF = 3
GN = GB * GNH
GALPHA = 0.2


def _leaky(v):
    return jnp.where(v >= 0, v, GALPHA * v)


def _dense_body(x_ref, w_ref, b_ref, o_ref):
    acc = jnp.dot(x_ref[...], w_ref[...], preferred_element_type=jnp.float32)
    o_ref[...] = _leaky(acc + b_ref[...])


def _dense(x, W, b):
    RB = 128
    return pl.pallas_call(
        _dense_body,
        grid=(GB // RB,),
        in_specs=[
            pl.BlockSpec((RB, GLD), lambda i: (i, 0)),
            pl.BlockSpec((GLD, GNH * GH0), lambda i: (0, 0)),
            pl.BlockSpec((1, GNH * GH0), lambda i: (0, 0)),
        ],
        out_specs=pl.BlockSpec((RB, GNH * GH0), lambda i: (i, 0)),
        out_shape=jax.ShapeDtypeStruct((GB, GNH * GH0), jnp.float32),
    )(x, W, b.reshape(1, -1))


def _conv_body(F, DO, GPB, pre_bn, *refs):
    if pre_bn:
        (st_ref, g_ref, bt_ref, h_ref, we_ref, be_ref, wr_ref, bias_ref,
         r_ref, s_ref, o_ref, so_ref, xs_scr, acc_scr) = refs
    else:
        (h_ref, we_ref, be_ref, wr_ref, bias_ref,
         r_ref, s_ref, o_ref, so_ref, xs_scr, acc_scr) = refs

    step = pl.program_id(0)
    nsteps = pl.num_programs(0)

    h = h_ref[...]
    if pre_bn:
        mean = st_ref[0:1, :] * (1.0 / GN)
        ex2 = st_ref[1:2, :] * (1.0 / GN)
        var = ex2 - mean * mean
        h = _leaky((h - mean) / jnp.sqrt(var + 1e-5) * g_ref[...] + bt_ref[...])

    ii = lax.broadcasted_iota(jnp.int32, (GNH, GNH), 1)
    jj = lax.broadcasted_iota(jnp.int32, (GNH, GNH), 0)
    eyeb = (ii == jj).astype(jnp.float32) * 1e10

    for g in range(GPB):
        hg = h[g * GNH:(g + 1) * GNH, :]
        sqg = jnp.sum(hg * hg, axis=1)
        gram = lax.dot_general(hg, hg, (((1,), (1,)), ((), ())),
                               preferred_element_type=jnp.float32)
        d2 = sqg[:, None] + sqg[None, :] - 2.0 * gram
        d2 = d2 + eyeb
        mrow = jnp.min(d2, axis=1, keepdims=True)
        tt = jnp.where(d2 == mrow, ii, GNH)
        nbr = jnp.min(tt, axis=1, keepdims=True)
        onehot = (ii == nbr).astype(jnp.float32)
        xs = lax.dot_general(onehot, hg, (((1,), (0,)), ((), ())),
                             preferred_element_type=jnp.float32,
                             precision=lax.Precision.HIGHEST)
        xs_scr[g * GNH:(g + 1) * GNH, :] = xs

    xs = xs_scr[...]
    ea = xs - h
    theta = jnp.dot(ea, we_ref[...], preferred_element_type=jnp.float32)
    theta = theta + be_ref[...]
    xbig = lax.dot_general(xs, r_ref[...], (((1,), (0,)), ((), ())),
                           preferred_element_type=jnp.float32,
                           precision=lax.Precision.HIGHEST)
    pr = (xbig.astype(jnp.bfloat16).astype(jnp.float32) *
          theta.astype(jnp.bfloat16).astype(jnp.float32))
    msg = lax.dot_general(pr, s_ref[...], (((1,), (0,)), ((), ())),
                          preferred_element_type=jnp.float32,
                          precision=lax.Precision.HIGHEST)
    out = jnp.dot(h, wr_ref[...], preferred_element_type=jnp.float32)
    out = out + msg + bias_ref[...]
    o_ref[...] = out

    @pl.when(step == 0)
    def _():
        acc_scr[...] = jnp.zeros_like(acc_scr)

    acc_scr[0:1, :] = acc_scr[0:1, :] + jnp.sum(out, axis=0, keepdims=True)
    acc_scr[1:2, :] = acc_scr[1:2, :] + jnp.sum(out * out, axis=0, keepdims=True)

    @pl.when(step == nsteps - 1)
    def _():
        so_ref[...] = acc_scr[...]


def _sel_mats(F, DO):
    R = np.zeros((F, F * DO), np.float32)
    S = np.zeros((F * DO, DO), np.float32)
    for i in range(F):
        R[i, i * DO:(i + 1) * DO] = 1.0
        S[i * DO:(i + 1) * DO, np.arange(DO)] = np.eye(DO)
    return jnp.asarray(R), jnp.asarray(S)


def _conv(h, We, be, Wr, bias, F, DO, pre_bn, bn_in=None):
    GPB = 8
    nb = GB // GPB
    R, S = _sel_mats(F, DO)
    body = functools.partial(_conv_body, F, DO, GPB, pre_bn)
    const = lambda i: (0, 0)
    in_specs = []
    args = []
    if pre_bn:
        st, gmm, bta = bn_in
        in_specs += [pl.BlockSpec((2, F), const),
                     pl.BlockSpec((1, F), const),
                     pl.BlockSpec((1, F), const)]
        args += [st, gmm.reshape(1, -1), bta.reshape(1, -1)]
    in_specs += [
        pl.BlockSpec((GPB * GNH, F), lambda i: (i, 0)),
        pl.BlockSpec((F, F * DO), const),
        pl.BlockSpec((1, F * DO), const),
        pl.BlockSpec((F, DO), const),
        pl.BlockSpec((1, DO), const),
        pl.BlockSpec((F, F * DO), const),
        pl.BlockSpec((F * DO, DO), const),
    ]
    args += [h, We, be.reshape(1, -1), Wr, bias.reshape(1, -1), R, S]
    return pl.pallas_call(
        body,
        grid=(nb,),
        in_specs=in_specs,
        out_specs=[pl.BlockSpec((GPB * GNH, DO), lambda i: (i, 0)),
                   pl.BlockSpec((2, DO), const)],
        out_shape=[jax.ShapeDtypeStruct((GN, DO), jnp.float32),
                   jax.ShapeDtypeStruct((2, DO), jnp.float32)],
        scratch_shapes=[pltpu.VMEM((GPB * GNH, F), jnp.float32),
                        pltpu.VMEM((2, DO), jnp.float32)],
    )(*args)


def _bnout_body(st_ref, g_ref, bt_ref, h_ref, o_ref):
    mean = st_ref[0:1, :] * (1.0 / GN)
    ex2 = st_ref[1:2, :] * (1.0 / GN)
    var = ex2 - mean * mean
    o_ref[...] = _leaky((h_ref[...] - mean) / jnp.sqrt(var + 1e-5)
                        * g_ref[...] + bt_ref[...])


def _bnout(h, st, gmm, bta, DO):
    RB = GN // 8
    const = lambda i: (0, 0)
    return pl.pallas_call(
        _bnout_body,
        grid=(8,),
        in_specs=[pl.BlockSpec((2, DO), const),
                  pl.BlockSpec((1, DO), const),
                  pl.BlockSpec((1, DO), const),
                  pl.BlockSpec((RB, DO), lambda i: (i, 0))],
        out_specs=pl.BlockSpec((RB, DO), lambda i: (i, 0)),
        out_shape=jax.ShapeDtypeStruct((GN, DO), jnp.float32),
    )(st, gmm.reshape(1, -1), bta.reshape(1, -1), h)


def kernel(x, W_dense, b_dense, W_edge1, b_edge1, W_root1, bias1, bn1_g,
           bn1_b, W_edge2, b_edge2, W_root2, bias2, bn2_g, bn2_b):
    h0 = _dense(x, W_dense, b_dense).reshape(GN, GH0)
    h1p, st1 = _conv(h0, W_edge1, b_edge1, W_root1, bias1, GH0, GH1,
                     pre_bn=False)
    h2p, st2 = _conv(h1p, W_edge2, b_edge2, W_root2, bias2, GH1, GNF,
                     pre_bn=True, bn_in=(st1, bn1_g, bn1_b))
    out = _bnout(h2p, st2, bn2_g, bn2_b, GNF)
    return out.reshape(GB, GNH, GNF)
